# baseline (speedup 1.0000x reference)
import functools

import jax
import jax.numpy as jnp
from jax import lax
from jax.experimental import pallas as pl
from jax.experimental.pallas import tpu as pltpu
from jax.experimental.pallas import tpu_sc as plsc

NC, NS = 2, 16
NB, NN, FD = 2, 10000, 128
TB = 80
NP = 10240
RPT = NP // NS
K = 128
GPB = NN // TB
BPP = NP // TB

_MESH = plsc.VectorSubcoreMesh(core_axis_name="c", subcore_axis_name="s")


def _deg_kernel(num_chunks):
    half = num_chunks // 2

    @functools.partial(
        pl.kernel,
        out_type=jax.ShapeDtypeStruct((NC * NP,), jnp.float32),
        mesh=_MESH,
        compiler_params=pltpu.CompilerParams(needs_layout_passes=False),
        scratch_types=[
            pltpu.VMEM((K,), jnp.int32),
            pltpu.VMEM((K,), jnp.int32),
            pltpu.VMEM((NP,), jnp.float32),
            pltpu.VMEM((RPT,), jnp.float32),
            pltpu.VMEM((RPT,), jnp.float32),
            pltpu.VMEM_SHARED((NS * NP,), jnp.float32),
            pltpu.SemaphoreType.DMA,
            pltpu.SemaphoreType.DMA,
        ],
    )
    def deg(dst_hbm, zeros_hbm, out_hbm, idx0_v, idx1_v, hist_v, tbuf_v,
            acc_v, stage_sh, isem0, isem1):
        idx_v = (idx0_v, idx1_v)
        isem = (isem0, isem1)
        core = lax.axis_index("c")
        wid = lax.axis_index("s")
        ones = jnp.ones((16,), jnp.float32)
        pltpu.async_copy(dst_hbm.at[wid * num_chunks + core], idx0_v, isem0)
        pltpu.sync_copy(zeros_hbm, hist_v)

        def chunk_body(k, carry):
            for p in range(2):
                kk = 2 * k + p
                pltpu.make_async_copy(dst_hbm.at[wid * num_chunks],
                                      idx_v[p], isem[p]).wait()

                @pl.when(kk + 1 < half)
                def _(kk=kk, p=p):
                    pltpu.async_copy(
                        dst_hbm.at[wid * num_chunks + core + 2 * (kk + 1)],
                        idx_v[1 - p], isem[1 - p])

                def vec_body(j, carry2, p=p):
                    idx = idx_v[p][pl.ds(j * 16, 16)]
                    plsc.addupdate_scatter(hist_v, [idx], ones)
                    return carry2

                lax.fori_loop(0, K // 16, vec_body, 0)
            return carry

        lax.fori_loop(0, half // 2, chunk_body, 0)
        pltpu.sync_copy(hist_v, stage_sh.at[pl.ds(wid * NP, NP)])
        plsc.subcore_barrier()

        pltpu.sync_copy(stage_sh.at[pl.ds(wid * RPT, RPT)], acc_v)

        def merge_body(t, carry):
            pltpu.sync_copy(stage_sh.at[pl.ds(t * NP + wid * RPT, RPT)], tbuf_v)

            def add_body(j, carry2):
                sl = pl.ds(j * 16, 16)
                acc_v[sl] = acc_v[sl] + tbuf_v[sl]
                return carry2

            lax.fori_loop(0, RPT // 16, add_body, 0)
            return carry

        lax.fori_loop(1, NS, merge_body, 0)
        pltpu.sync_copy(acc_v, out_hbm.at[pl.ds(core * NP + wid * RPT, RPT)])

    return deg


NBUF = 2


FDX = FD


def _scatter_kernel(num_chunks):
    assert num_chunks % NBUF == 0

    @functools.partial(
        pl.kernel,
        out_type=jax.ShapeDtypeStruct((NC * NP, FDX), jnp.float32),
        mesh=_MESH,
        scratch_types=(
            [pltpu.VMEM((K,), jnp.int32)] * (2 * NBUF)
            + [pltpu.VMEM((K, FDX), jnp.float32)] * NBUF
            + [pltpu.VMEM_SHARED((NP, FDX), jnp.float32)]
            + [pltpu.SemaphoreType.DMA] * (2 * NBUF)
        ),
    )
    def scat(g_hbm, srcg_hbm, dstl_hbm, zeros_hbm, out_hbm, *sc):
        src_v = sc[0:NBUF]
        dst_v = sc[NBUF:2 * NBUF]
        rows_v = sc[2 * NBUF:3 * NBUF]
        acc_sh = sc[3 * NBUF]
        gsem = sc[3 * NBUF + 1:4 * NBUF + 1]
        isem = sc[4 * NBUF + 1:5 * NBUF + 1]
        core = lax.axis_index("c")
        wid = lax.axis_index("s")
        srow = (core * NS + wid) * num_chunks
        drow = wid * num_chunks
        pltpu.sync_copy(zeros_hbm.at[pl.ds(wid * RPT, RPT)],
                        acc_sh.at[pl.ds(wid * RPT, RPT)])
        plsc.subcore_barrier()

        for j in range(NBUF - 1):
            pltpu.sync_copy(srcg_hbm.at[srow + j], src_v[j])
            pltpu.sync_copy(dstl_hbm.at[drow + j], dst_v[j])
            pltpu.async_copy(g_hbm.at[src_v[j]], rows_v[j], gsem[j])
        last = NBUF - 1
        pltpu.async_copy(srcg_hbm.at[srow + last], src_v[last], isem[last])
        pltpu.async_copy(dstl_hbm.at[drow + last], dst_v[last], isem[last])

        def body(k, carry):
            base = k * NBUF
            for p in range(NBUF):
                c = base + p
                q = (p + NBUF - 1) % NBUF
                @pl.when(c + NBUF - 1 < num_chunks)
                def _(p=p, q=q):
                    pltpu.make_async_copy(srcg_hbm.at[srow], src_v[q],
                                          isem[q]).wait()
                    pltpu.make_async_copy(dstl_hbm.at[drow], dst_v[q],
                                          isem[q]).wait()
                    pltpu.async_copy(g_hbm.at[src_v[q]], rows_v[q], gsem[q])

                pltpu.make_async_copy(g_hbm.at[src_v[p]], rows_v[p],
                                      gsem[p]).wait()
                pltpu.sync_copy(rows_v[p], acc_sh.at[dst_v[p]], add=True)

                @pl.when(c + NBUF < num_chunks)
                def _(c=c, p=p):
                    pltpu.async_copy(srcg_hbm.at[srow + c + NBUF], src_v[p],
                                     isem[p])
                    pltpu.async_copy(dstl_hbm.at[drow + c + NBUF], dst_v[p],
                                     isem[p])
            return carry

        lax.fori_loop(0, num_chunks // NBUF, body, 0)
        plsc.subcore_barrier()
        pltpu.sync_copy(acc_sh.at[pl.ds(wid * RPT, RPT)],
                        out_hbm.at[pl.ds(core * NP + wid * RPT, RPT)])

    return scat


def _acc_map(i):
    return (i // GPB * BPP + i % GPB, 0)


def _row_spec():
    return pl.BlockSpec((TB, FD), lambda i: (i, 0))


def _deg_specs():
    return [
        pl.BlockSpec((TB, 1), lambda i: (i % GPB, 0)),
        pl.BlockSpec((TB, 1), lambda i: (NP // TB + i % GPB, 0)),
    ]


def _dinv(dga_ref, dgb_ref):
    return lax.rsqrt(dga_ref[...] + dgb_ref[...] + 1.0)


def _tc1_body(x_ref, w_ref, dga_ref, dgb_ref, g_ref):
    h = jnp.dot(x_ref[...], w_ref[...], preferred_element_type=jnp.float32)
    g_ref[...] = h * _dinv(dga_ref, dgb_ref)


def _tc2_body(acc_ref, g1_ref, w_ref, b_ref, dga_ref, dgb_ref, g2_ref):
    dinv = _dinv(dga_ref, dgb_ref)
    h = jnp.maximum(dinv * (acc_ref[...] + g1_ref[...]) + b_ref[...], 0.0)
    g2_ref[...] = jnp.dot(h, w_ref[...], preferred_element_type=jnp.float32) * dinv


def _tc3_body(acc_ref, g2_ref, b_ref, gam_ref, bet_ref, dga_ref, dgb_ref,
              o_ref):
    dinv = _dinv(dga_ref, dgb_ref)
    t = dinv * (acc_ref[...] + g2_ref[...]) + b_ref[...]
    mu = jnp.mean(t, axis=1, keepdims=True)
    d = t - mu
    var = jnp.mean(d * d, axis=1, keepdims=True)
    o_ref[...] = d * lax.rsqrt(var + 1e-5) * gam_ref[...] + bet_ref[...]


def kernel(x, edge_index, W1, b1, W2, b2, gamma, beta):
    nb, nn, fd = x.shape
    x2d = x.reshape(nb * nn, fd)
    e = edge_index.shape[1]
    src = edge_index[0].astype(jnp.int32)
    dst = edge_index[1].astype(jnp.int32)

    et = -(-e // NS)
    c = -(-et // (K * 4)) * 4
    etp = c * K
    srcp = jnp.pad(src, (0, NS * etp - e)).reshape(NS, etp)
    dstp = jnp.pad(dst, (0, NS * etp - e), constant_values=NN).reshape(NS, etp)
    src_rows = jnp.stack([srcp, srcp + NN]).reshape(NC * NS * c, K)
    dst_rows = dstp.reshape(NS * c, K)

    zeros1d = jnp.zeros((NP,), jnp.float32)
    zeros_acc = jnp.zeros((NP, FD), jnp.float32)

    deg_p = _deg_kernel(c)(dst_rows, zeros1d).reshape(NC * NP, 1)

    grid = (nb * nn // TB,)
    g1 = pl.pallas_call(
        _tc1_body,
        grid=grid,
        in_specs=[
            _row_spec(),
            pl.BlockSpec((FD, FD), lambda i: (0, 0)),
            *_deg_specs(),
        ],
        out_specs=_row_spec(),
        out_shape=jax.ShapeDtypeStruct((nb * nn, FD), jnp.float32),
    )(x2d, W1, deg_p, deg_p)

    scat = _scatter_kernel(c)
    acc1 = scat(g1, src_rows, dst_rows, zeros_acc)

    g2 = pl.pallas_call(
        _tc2_body,
        grid=grid,
        in_specs=[
            pl.BlockSpec((TB, FD), _acc_map),
            _row_spec(),
            pl.BlockSpec((FD, FD), lambda i: (0, 0)),
            pl.BlockSpec((1, FD), lambda i: (0, 0)),
            *_deg_specs(),
        ],
        out_specs=_row_spec(),
        out_shape=jax.ShapeDtypeStruct((nb * nn, FD), jnp.float32),
    )(acc1, g1, W2, b1.reshape(1, FD), deg_p, deg_p)

    acc2 = scat(g2, src_rows, dst_rows, zeros_acc)

    out = pl.pallas_call(
        _tc3_body,
        grid=grid,
        in_specs=[
            pl.BlockSpec((TB, FD), _acc_map),
            _row_spec(),
            pl.BlockSpec((1, FD), lambda i: (0, 0)),
            pl.BlockSpec((1, FD), lambda i: (0, 0)),
            pl.BlockSpec((1, FD), lambda i: (0, 0)),
            *_deg_specs(),
        ],
        out_specs=_row_spec(),
        out_shape=jax.ShapeDtypeStruct((nb * nn, FD), jnp.float32),
    )(acc2, g2, b2.reshape(1, FD), gamma.reshape(1, FD), beta.reshape(1, FD),
      deg_p, deg_p)

    return out.reshape(nb, nn, FD)


# speedup vs baseline: 1.3460x; 1.3460x over previous
import functools

import jax
import jax.numpy as jnp
from jax import lax
from jax.experimental import pallas as pl
from jax.experimental.pallas import tpu as pltpu
from jax.experimental.pallas import tpu_sc as plsc

NC, NS = 2, 16
NB, NN, FD = 2, 10000, 128
TB = 80
NP = 10240
RPT = NP // NS
K = 128
GPB = NN // TB
BPP = NP // TB

_MESH = plsc.VectorSubcoreMesh(core_axis_name="c", subcore_axis_name="s")


def _deg_kernel(num_chunks):
    @functools.partial(
        pl.kernel,
        out_type=jax.ShapeDtypeStruct((NC * NP,), jnp.float32),
        mesh=_MESH,
        compiler_params=pltpu.CompilerParams(needs_layout_passes=False),
        scratch_types=[
            pltpu.VMEM((num_chunks * K,), jnp.int32),
            pltpu.VMEM((NP,), jnp.float32),
            pltpu.VMEM((RPT,), jnp.float32),
            pltpu.VMEM((RPT,), jnp.float32),
            pltpu.VMEM_SHARED((NS * NP,), jnp.float32),
        ],
    )
    def deg(dst_hbm, zeros_hbm, out_hbm, idx_v, hist_v, tbuf_v, acc_v, stage_sh):
        core = lax.axis_index("c")
        wid = lax.axis_index("s")
        n = num_chunks * K
        pltpu.sync_copy(dst_hbm.at[pl.ds(wid * n, n)], idx_v)
        pltpu.sync_copy(zeros_hbm, hist_v)
        ones = jnp.ones((16,), jnp.float32)

        def vec_body(j, carry):
            idx = idx_v[pl.ds(j * 16, 16)]
            plsc.addupdate_scatter(hist_v, [idx], ones)
            return carry

        lax.fori_loop(0, n // 16, vec_body, 0)
        pltpu.sync_copy(hist_v, stage_sh.at[pl.ds(wid * NP, NP)])
        plsc.subcore_barrier()

        pltpu.sync_copy(stage_sh.at[pl.ds(wid * RPT, RPT)], acc_v)

        def merge_body(t, carry):
            pltpu.sync_copy(stage_sh.at[pl.ds(t * NP + wid * RPT, RPT)], tbuf_v)

            def add_body(j, carry2):
                sl = pl.ds(j * 16, 16)
                acc_v[sl] = acc_v[sl] + tbuf_v[sl]
                return carry2

            lax.fori_loop(0, RPT // 16, add_body, 0)
            return carry

        lax.fori_loop(1, NS, merge_body, 0)
        pltpu.sync_copy(acc_v, out_hbm.at[pl.ds(core * NP + wid * RPT, RPT)])

    return deg


NBUF = 2


FDX = FD


def _scatter_kernel(num_chunks):
    assert num_chunks % NBUF == 0

    @functools.partial(
        pl.kernel,
        out_type=jax.ShapeDtypeStruct((NC * NP, FDX), jnp.float32),
        mesh=_MESH,
        scratch_types=(
            [pltpu.VMEM((K,), jnp.int32)] * (2 * NBUF)
            + [pltpu.VMEM((K, FDX), jnp.float32)] * NBUF
            + [pltpu.VMEM_SHARED((NP, FDX), jnp.float32)]
            + [pltpu.SemaphoreType.DMA] * (2 * NBUF)
        ),
    )
    def scat(g_hbm, srcg_hbm, dstl_hbm, zeros_hbm, out_hbm, *sc):
        src_v = sc[0:NBUF]
        dst_v = sc[NBUF:2 * NBUF]
        rows_v = sc[2 * NBUF:3 * NBUF]
        acc_sh = sc[3 * NBUF]
        gsem = sc[3 * NBUF + 1:4 * NBUF + 1]
        isem = sc[4 * NBUF + 1:5 * NBUF + 1]
        core = lax.axis_index("c")
        wid = lax.axis_index("s")
        srow = (core * NS + wid) * num_chunks
        drow = wid * num_chunks
        pltpu.sync_copy(zeros_hbm.at[pl.ds(wid * RPT, RPT)],
                        acc_sh.at[pl.ds(wid * RPT, RPT)])
        plsc.subcore_barrier()

        for j in range(NBUF - 1):
            pltpu.sync_copy(srcg_hbm.at[srow + j], src_v[j])
            pltpu.sync_copy(dstl_hbm.at[drow + j], dst_v[j])
            pltpu.async_copy(g_hbm.at[src_v[j]], rows_v[j], gsem[j])
        last = NBUF - 1
        pltpu.async_copy(srcg_hbm.at[srow + last], src_v[last], isem[last])
        pltpu.async_copy(dstl_hbm.at[drow + last], dst_v[last], isem[last])

        def body(k, carry):
            base = k * NBUF
            for p in range(NBUF):
                c = base + p
                q = (p + NBUF - 1) % NBUF
                @pl.when(c + NBUF - 1 < num_chunks)
                def _(p=p, q=q):
                    pltpu.make_async_copy(srcg_hbm.at[srow], src_v[q],
                                          isem[q]).wait()
                    pltpu.make_async_copy(dstl_hbm.at[drow], dst_v[q],
                                          isem[q]).wait()
                    pltpu.async_copy(g_hbm.at[src_v[q]], rows_v[q], gsem[q])

                pltpu.make_async_copy(g_hbm.at[src_v[p]], rows_v[p],
                                      gsem[p]).wait()
                pltpu.sync_copy(rows_v[p], acc_sh.at[dst_v[p]], add=True)

                @pl.when(c + NBUF < num_chunks)
                def _(c=c, p=p):
                    pltpu.async_copy(srcg_hbm.at[srow + c + NBUF], src_v[p],
                                     isem[p])
                    pltpu.async_copy(dstl_hbm.at[drow + c + NBUF], dst_v[p],
                                     isem[p])
            return carry

        lax.fori_loop(0, num_chunks // NBUF, body, 0)
        plsc.subcore_barrier()
        pltpu.sync_copy(acc_sh.at[pl.ds(wid * RPT, RPT)],
                        out_hbm.at[pl.ds(core * NP + wid * RPT, RPT)])

    return scat


def _acc_map(i):
    return (i // GPB * BPP + i % GPB, 0)


def _row_spec():
    return pl.BlockSpec((TB, FD), lambda i: (i, 0))


def _deg_spec():
    return pl.BlockSpec((TB, 1), lambda i: (i, 0))


def _dinv(dg_ref):
    return lax.rsqrt(dg_ref[...] + 1.0)


def _tc1_body(x_ref, w_ref, dg_ref, g_ref):
    h = jnp.dot(x_ref[...], w_ref[...], preferred_element_type=jnp.float32)
    g_ref[...] = h * _dinv(dg_ref)


def _tc2_body(acc_ref, g1_ref, w_ref, b_ref, dg_ref, g2_ref):
    dinv = _dinv(dg_ref)
    h = jnp.maximum(dinv * (acc_ref[...] + g1_ref[...]) + b_ref[...], 0.0)
    g2_ref[...] = jnp.dot(h, w_ref[...], preferred_element_type=jnp.float32) * dinv


def _tc3_body(acc_ref, g2_ref, b_ref, gam_ref, bet_ref, dg_ref, o_ref):
    dinv = _dinv(dg_ref)
    t = dinv * (acc_ref[...] + g2_ref[...]) + b_ref[...]
    mu = jnp.mean(t, axis=1, keepdims=True)
    d = t - mu
    var = jnp.mean(d * d, axis=1, keepdims=True)
    o_ref[...] = d * lax.rsqrt(var + 1e-5) * gam_ref[...] + bet_ref[...]


def kernel(x, edge_index, W1, b1, W2, b2, gamma, beta):
    nb, nn, fd = x.shape
    x2d = x.reshape(nb * nn, fd)
    e = edge_index.shape[1]
    src = edge_index[0].astype(jnp.int32)
    dst = edge_index[1].astype(jnp.int32)

    et = -(-e // NS)
    c = -(-et // (K * NBUF)) * NBUF
    etp = c * K
    srcp = jnp.pad(src, (0, NS * etp - e)).reshape(NS, etp)
    dstp = jnp.pad(dst, (0, NS * etp - e), constant_values=NN).reshape(NS, etp)
    src_rows = jnp.stack([srcp, srcp + NN]).reshape(NC * NS * c, K)
    dst_rows = dstp.reshape(NS * c, K)

    zeros1d = jnp.zeros((NP,), jnp.float32)
    zeros_acc = jnp.zeros((NP, FD), jnp.float32)

    deg_p = _deg_kernel(c)(dstp.reshape(NS * etp), zeros1d)
    deg_col = jnp.concatenate([deg_p[:nn], deg_p[:nn]]).reshape(nb * nn, 1)

    grid = (nb * nn // TB,)
    g1 = pl.pallas_call(
        _tc1_body,
        grid=grid,
        in_specs=[
            _row_spec(),
            pl.BlockSpec((FD, FD), lambda i: (0, 0)),
            _deg_spec(),
        ],
        out_specs=_row_spec(),
        out_shape=jax.ShapeDtypeStruct((nb * nn, FD), jnp.float32),
    )(x2d, W1, deg_col)

    scat = _scatter_kernel(c)
    acc1 = scat(g1, src_rows, dst_rows, zeros_acc)

    g2 = pl.pallas_call(
        _tc2_body,
        grid=grid,
        in_specs=[
            pl.BlockSpec((TB, FD), _acc_map),
            _row_spec(),
            pl.BlockSpec((FD, FD), lambda i: (0, 0)),
            pl.BlockSpec((1, FD), lambda i: (0, 0)),
            _deg_spec(),
        ],
        out_specs=_row_spec(),
        out_shape=jax.ShapeDtypeStruct((nb * nn, FD), jnp.float32),
    )(acc1, g1, W2, b1.reshape(1, FD), deg_col)

    acc2 = scat(g2, src_rows, dst_rows, zeros_acc)

    out = pl.pallas_call(
        _tc3_body,
        grid=grid,
        in_specs=[
            pl.BlockSpec((TB, FD), _acc_map),
            _row_spec(),
            pl.BlockSpec((1, FD), lambda i: (0, 0)),
            pl.BlockSpec((1, FD), lambda i: (0, 0)),
            pl.BlockSpec((1, FD), lambda i: (0, 0)),
            _deg_spec(),
        ],
        out_specs=_row_spec(),
        out_shape=jax.ShapeDtypeStruct((nb * nn, FD), jnp.float32),
    )(acc2, g2, b2.reshape(1, FD), gamma.reshape(1, FD), beta.reshape(1, FD),
      deg_col)

    return out.reshape(nb, nn, FD)
